# baseline (speedup 1.0000x reference)
import numpy as np
import jax
import jax.numpy as jnp
from jax.experimental import pallas as pl

_B = 128
_N = 100000
_RB = 8
_GRID = _B // _RB
_S = 1024
_PHYS = 100096
_LAST = _PHYS - _S
_STRIPS = _N // _S + 1

_KEY_HI = np.uint32(0)
_KEY_LO = np.uint32(42)
_KS2 = np.uint32(_KEY_HI ^ _KEY_LO ^ np.uint32(0x1BD11BDA))
_ROTS = ((13, 15, 26, 6), (17, 29, 16, 24))
_TINY = np.float32(np.finfo(np.float32).tiny)
_INT_MAX = np.int32(2**31 - 1)
_NEG_INF = np.float32(-np.inf)


def _rotl(x, d):
    return (x << np.uint32(d)) | (x >> np.uint32(32 - d))


def _threefry2x32(x0, x1):
    ks = (_KEY_HI, _KEY_LO, _KS2)
    x0 = x0 + ks[0]
    x1 = x1 + ks[1]
    for i in range(5):
        for r in _ROTS[i % 2]:
            x0 = x0 + x1
            x1 = _rotl(x1, r)
            x1 = x0 ^ x1
        x0 = x0 + ks[(i + 1) % 3]
        x1 = x1 + np.uint32(ks[(i + 2) % 3] + np.uint32(i + 1))
    return x0, x1


def _body(logits_ref, out_ref):
    i = pl.program_id(0)
    shape = (_RB, _S)
    lane = jax.lax.broadcasted_iota(jnp.int32, shape, 1)
    row_flat = (jax.lax.broadcasted_iota(jnp.uint32, shape, 0)
                + (i * _RB).astype(jnp.uint32)) * np.uint32(_N)

    def strip(k, carry):
        bm, bc = carry
        col0 = jnp.minimum(k * _S, _LAST)
        gcol = col0 + lane
        flat = row_flat + gcol.astype(jnp.uint32)
        v0, v1 = _threefry2x32(jnp.zeros(shape, jnp.uint32), flat)
        bits = v0 ^ v1
        float_bits = (bits >> np.uint32(9)) | np.uint32(0x3F800000)
        frac = jax.lax.bitcast_convert_type(float_bits, jnp.float32) - np.float32(1.0)
        u = jnp.maximum(_TINY, frac)
        vals = logits_ref[:, pl.ds(col0, _S)] - jnp.log(-jnp.log(u))
        vals = jnp.where(gcol < _N, vals, _NEG_INF)
        better = vals > bm
        bm = jnp.where(better, vals, bm)
        bc = jnp.where(better, gcol, bc)
        return bm, bc

    bm, bc = jax.lax.fori_loop(
        0, _STRIPS, strip,
        (jnp.full(shape, _NEG_INF, jnp.float32), jnp.zeros(shape, jnp.int32)),
        unroll=7,
    )

    m = jnp.max(bm, axis=1, keepdims=True)
    out_ref[...] = jnp.min(jnp.where(bm == m, bc, _INT_MAX), axis=1, keepdims=True)


def kernel(logits):
    out = pl.pallas_call(
        _body,
        grid=(_GRID,),
        in_specs=[pl.BlockSpec((_RB, _N), lambda i: (i, 0))],
        out_specs=pl.BlockSpec((_RB, 1), lambda i: (i, 0)),
        out_shape=jax.ShapeDtypeStruct((_B, 1), jnp.int32),
    )(logits)
    return out.reshape(_B)


# speedup vs baseline: 1.2534x; 1.2534x over previous
import numpy as np
import jax
import jax.numpy as jnp
from jax.experimental import pallas as pl
from jax.experimental.pallas import tpu as pltpu

_B = 128
_N = 100000
_CHUNK = 10000
_GRID = _N // _CHUNK
_SR = 80
_STRIPS = _CHUNK // _SR

_KEY_HI = np.uint32(0)
_KEY_LO = np.uint32(42)
_KS2 = np.uint32(_KEY_HI ^ _KEY_LO ^ np.uint32(0x1BD11BDA))
_ROTS = ((13, 15, 26, 6), (17, 29, 16, 24))
_TINY = np.float32(np.finfo(np.float32).tiny)
_INT_MAX = np.int32(2**31 - 1)
_NEG_INF = np.float32(-np.inf)


def _rotl(x, d):
    return (x << np.uint32(d)) | (x >> np.uint32(32 - d))


def _threefry2x32(x0, x1):
    ks = (_KEY_HI, _KEY_LO, _KS2)
    x0 = x0 + ks[0]
    x1 = x1 + ks[1]
    for i in range(5):
        for r in _ROTS[i % 2]:
            x0 = x0 + x1
            x1 = _rotl(x1, r)
            x1 = x0 ^ x1
        x0 = x0 + ks[(i + 1) % 3]
        x1 = x1 + np.uint32(ks[(i + 2) % 3] + np.uint32(i + 1))
    return x0, x1


def _body(logits_ref, out_ref, best_val):
    j = pl.program_id(0)
    shape = (_SR, 128)
    lane_mul = jax.lax.broadcasted_iota(jnp.uint32, shape, 1) * np.uint32(_N)
    sub = jax.lax.broadcasted_iota(jnp.int32, shape, 0)

    def strip(k, carry):
        bm, bc = carry
        row0 = k * _SR
        c = j * _CHUNK + row0 + sub
        flat = lane_mul + c.astype(jnp.uint32)
        v0, v1 = _threefry2x32(jnp.zeros(shape, jnp.uint32), flat)
        bits = v0 ^ v1
        float_bits = (bits >> np.uint32(9)) | np.uint32(0x3F800000)
        frac = jax.lax.bitcast_convert_type(float_bits, jnp.float32) - np.float32(1.0)
        u = jnp.maximum(_TINY, frac)
        vals = logits_ref[pl.ds(row0, _SR), :] - jnp.log(-jnp.log(u))
        better = vals > bm
        bm = jnp.where(better, vals, bm)
        bc = jnp.where(better, c, bc)
        return bm, bc

    bm, bc = jax.lax.fori_loop(
        0, _STRIPS, strip,
        (jnp.full(shape, _NEG_INF, jnp.float32), jnp.zeros(shape, jnp.int32)),
        unroll=5,
    )

    m = jnp.max(bm, axis=0, keepdims=True)
    idx = jnp.min(jnp.where(bm == m, bc, _INT_MAX), axis=0, keepdims=True)

    @pl.when(j == 0)
    def _():
        best_val[...] = m
        out_ref[...] = idx

    @pl.when(j > 0)
    def _():
        bv = best_val[...]
        better = m > bv
        best_val[...] = jnp.where(better, m, bv)
        out_ref[...] = jnp.where(better, idx, out_ref[...])


def kernel(logits):
    out = pl.pallas_call(
        _body,
        grid=(_GRID,),
        in_specs=[pl.BlockSpec((_CHUNK, _B), lambda j: (j, 0))],
        out_specs=pl.BlockSpec((1, _B), lambda j: (0, 0)),
        out_shape=jax.ShapeDtypeStruct((1, _B), jnp.int32),
        scratch_shapes=[pltpu.VMEM((1, _B), jnp.float32)],
    )(logits.T)
    return out.reshape(_B)
